# baseline (speedup 1.0000x reference)
import functools

import jax
import jax.numpy as jnp
from jax import lax
from jax.experimental import pallas as pl
from jax.experimental.pallas import tpu as pltpu
from jax.experimental.pallas import tpu_sc as plsc

V = 1000000
D = 64
B = 16384

NC = 2
NS = 16
NW = NC * NS
B_PER_W = B // NW

_MESH = plsc.VectorSubcoreMesh(core_axis_name="c", subcore_axis_name="s")


@functools.partial(
    pl.kernel,
    mesh=_MESH,
    out_type=jax.ShapeDtypeStruct((B, D), jnp.float32),
    scratch_types=[
        pltpu.VMEM((B_PER_W,), jnp.int32),
        pltpu.VMEM((B_PER_W, D), jnp.float32),
        pltpu.SemaphoreType.DMA,
    ],
    compiler_params=pltpu.CompilerParams(use_tc_tiling_on_sc=False),
)
def _sc_gather(idx_hbm, table_hbm, out_hbm, idx_v, rows_v, sem):
    wid = lax.axis_index("s") * NC + lax.axis_index("c")
    base = wid * B_PER_W
    pltpu.sync_copy(idx_hbm.at[pl.ds(base, B_PER_W)], idx_v)
    pltpu.async_copy(table_hbm.at[idx_v], rows_v, sem).wait()
    pltpu.sync_copy(rows_v, out_hbm.at[pl.ds(base, B_PER_W)])


def _ls_body(x_ref, o_ref):
    x = x_ref[...]
    m = jnp.max(x, axis=1, keepdims=True)
    sh = x - m
    s = jnp.sum(jnp.exp(sh), axis=1, keepdims=True)
    o_ref[...] = sh - jnp.log(s)


def _tc_log_softmax(x):
    blk = 2048
    return pl.pallas_call(
        _ls_body,
        out_shape=jax.ShapeDtypeStruct((B, D), jnp.float32),
        grid=(B // blk,),
        in_specs=[pl.BlockSpec((blk, D), lambda i: (i, 0))],
        out_specs=pl.BlockSpec((blk, D), lambda i: (i, 0)),
    )(x)


@jax.jit
def kernel(input_vec, W):
    rows = _sc_gather(input_vec, W)
    return _tc_log_softmax(rows)


# speedup vs baseline: 1.6035x; 1.6035x over previous
import functools

import jax
import jax.numpy as jnp
from jax import lax
from jax.experimental import pallas as pl
from jax.experimental.pallas import tpu as pltpu
from jax.experimental.pallas import tpu_sc as plsc

V = 1000000
D = 64
B = 16384

NC = 2
NS = 16
NW = NC * NS
B_PER_W = B // NW
G = 16
NG = B_PER_W // G

_LN2 = 0.6931471805599453

_MESH = plsc.VectorSubcoreMesh(core_axis_name="c", subcore_axis_name="s")


def _log_vec(s):
    bits = plsc.bitcast(s, jnp.int32)
    e = (bits >> 23) - 127
    mbits = (bits & 0x007FFFFF) | 0x3F800000
    m = plsc.bitcast(mbits, jnp.float32)
    big = m > 1.5
    m = jnp.where(big, m * 0.5, m)
    e = e + jnp.where(big, 1, 0)
    t = (m - 1.0) / (m + 1.0)
    t2 = t * t
    ln_m = t * (2.0 + t2 * (2.0 / 3.0 + t2 * (2.0 / 5.0 + t2 * (2.0 / 7.0))))
    return e.astype(jnp.float32) * _LN2 + ln_m


@functools.partial(
    pl.kernel,
    mesh=_MESH,
    out_type=jax.ShapeDtypeStruct((D, B), jnp.float32),
    scratch_types=[
        pltpu.VMEM((B_PER_W,), jnp.int32),
        pltpu.VMEM((G, D), jnp.float32),
        pltpu.VMEM((G, D), jnp.float32),
        pltpu.VMEM((D, B_PER_W), jnp.float32),
        pltpu.SemaphoreType.DMA,
        pltpu.SemaphoreType.DMA,
    ],
    compiler_params=pltpu.CompilerParams(
        use_tc_tiling_on_sc=True, needs_layout_passes=False
    ),
)
def _sc_fused(idx_hbm, w_hbm, out_hbm, idx_v, st0, st1, ost, sem0, sem1):
    wid = lax.axis_index("s") * NC + lax.axis_index("c")
    base = wid * B_PER_W
    pltpu.sync_copy(idx_hbm.at[pl.ds(base, B_PER_W)], idx_v)

    rows16 = lax.iota(jnp.int32, 16)

    def fire(g, st, sem):
        iv = idx_v[pl.ds(g * G, G)]
        for j in range(G):
            r = iv[j]
            pltpu.async_copy(
                w_hbm.at[pl.ds(r, 1), :], st.at[pl.ds(j, 1), :], sem
            )

    def drain(st, sem):
        pltpu.make_async_copy(
            w_hbm.at[pl.ds(0, G), :], st.at[pl.ds(0, G), :], sem
        ).wait()

    def compute(g, st):
        def col(c):
            return plsc.load_gather(st, [rows16, jnp.full((16,), c, jnp.int32)])

        m = col(0)
        for c in range(1, D):
            m = jnp.maximum(m, col(c))
        s = jnp.exp(col(0) - m)
        for c in range(1, D):
            s = s + jnp.exp(col(c) - m)
        tot = m + _log_vec(s)
        k = g * G + rows16
        for c in range(D):
            plsc.store_scatter(
                ost, [jnp.full((16,), c, jnp.int32), k], col(c) - tot
            )

    fire(0, st0, sem0)

    def body(h, _):
        g0 = 2 * h
        fire(g0 + 1, st1, sem1)
        drain(st0, sem0)
        compute(g0, st0)

        @pl.when(h < NG // 2 - 1)
        def _():
            fire(g0 + 2, st0, sem0)

        drain(st1, sem1)
        compute(g0 + 1, st1)
        return 0

    lax.fori_loop(0, NG // 2, body, 0)

    pltpu.sync_copy(ost, out_hbm.at[:, pl.ds(base, B_PER_W)])


@jax.jit
def kernel(input_vec, W):
    outT = _sc_fused(input_vec, W)
    return outT.T
